# Initial kernel scaffold; baseline (speedup 1.0000x reference)
import jax
import jax.numpy as jnp
from jax.experimental import pallas as pl


def kernel(x, edge_index, edge_label_index, W1, b1, W2, b2, Wm1, bm1, Wm2, bm2):
    raise NotImplementedError("write your pallas kernel here")


# speedup vs baseline: 11.5593x; 11.5593x over previous
import functools

import jax
import jax.numpy as jnp
from jax import lax
from jax.experimental import pallas as pl
from jax.experimental.pallas import tpu as pltpu
from jax.experimental.pallas import tpu_sc as plsc

F32 = jnp.float32

_NC = 2
_NS = 16
_NW = _NC * _NS


def _sc_mesh():
    return plsc.VectorSubcoreMesh(core_axis_name="c", subcore_axis_name="s")


def _pad_nodes(n):
    return ((n + 8 * _NS - 1) // (8 * _NS)) * (8 * _NS)


def _degree_partials(dst, n_nodes):
    (e,) = dst.shape
    ept = e // _NW
    B = 200
    assert ept % B == 0 and B % 8 == 0
    nb = ept // B
    np_ = _pad_nodes(n_nodes)
    rpt = np_ // _NS
    assert (rpt % B) % 8 == 0
    W = 128

    @functools.partial(
        pl.kernel,
        out_type=jax.ShapeDtypeStruct((2 * np_, W), F32),
        mesh=_sc_mesh(),
        scratch_types=[
            pltpu.VMEM((B,), jnp.int32),
            pltpu.VMEM((B, W), F32),
            pltpu.VMEM_SHARED((np_, W), F32),
        ],
    )
    def k(dst_h, out_h, idx_v, ones_v, acc_sh):
        c = lax.axis_index("c")
        s = lax.axis_index("s")

        def fill(val):
            def row(i, _):
                for t in range(W // 16):
                    ones_v[i, pl.ds(16 * t, 16)] = jnp.full((16,), val, F32)
                return 0
            lax.fori_loop(0, B, row, 0)

        fill(0.0)
        for j in range(rpt // B):
            pltpu.sync_copy(ones_v, acc_sh.at[pl.ds(s * rpt + j * B, B)])
        rem = rpt % B
        if rem:
            pltpu.sync_copy(ones_v.at[pl.ds(0, rem)],
                            acc_sh.at[pl.ds(s * rpt + rpt - rem, rem)])
        fill(1.0)
        plsc.subcore_barrier()

        ebase = c * (e // _NC) + s * ept

        def body(b, _):
            off = ebase + b * B
            pltpu.sync_copy(dst_h.at[pl.ds(off, B)], idx_v)
            pltpu.sync_copy(ones_v, acc_sh.at[idx_v], add=True)
            return 0

        lax.fori_loop(0, nb, body, 0)
        plsc.subcore_barrier()
        pltpu.sync_copy(acc_sh.at[pl.ds(s * rpt, rpt)],
                        out_h.at[pl.ds(c * np_ + s * rpt, rpt)])

    return k(dst)


def _msg_partials(xs, src, dst):
    n, d = xs.shape
    (e,) = src.shape
    ept = e // _NW
    B = 200 if d > 64 else 400
    assert ept % B == 0 and B % 8 == 0
    nb = ept // B
    np_ = _pad_nodes(n)
    rpt = np_ // _NS
    assert (rpt % B) % 8 == 0
    nch = d // 16

    @functools.partial(
        pl.kernel,
        out_type=jax.ShapeDtypeStruct((2 * np_, d), F32),
        mesh=_sc_mesh(),
        scratch_types=[
            pltpu.VMEM((B,), jnp.int32),
            pltpu.VMEM((B,), jnp.int32),
            pltpu.VMEM((B, d), F32),
            pltpu.VMEM_SHARED((np_, d), F32),
            pltpu.SemaphoreType.DMA,
        ],
    )
    def k(xs_h, src_h, dst_h, out_h, src_v, dst_v, rows_v, acc_sh, sem):
        c = lax.axis_index("c")
        s = lax.axis_index("s")

        def zrow(i, _):
            for t in range(nch):
                rows_v[i, pl.ds(16 * t, 16)] = jnp.zeros((16,), F32)
            return 0

        lax.fori_loop(0, B, zrow, 0)
        r0 = s * rpt
        for j in range(rpt // B):
            pltpu.sync_copy(rows_v, acc_sh.at[pl.ds(r0 + j * B, B)])
        rem = rpt % B
        if rem:
            pltpu.sync_copy(rows_v.at[pl.ds(0, rem)],
                            acc_sh.at[pl.ds(r0 + rpt - rem, rem)])
        plsc.subcore_barrier()

        ebase = c * (e // _NC) + s * ept

        def body(b, _):
            off = ebase + b * B
            pltpu.sync_copy(src_h.at[pl.ds(off, B)], src_v)
            gp = pltpu.async_copy(xs_h.at[src_v], rows_v, sem)
            pltpu.sync_copy(dst_h.at[pl.ds(off, B)], dst_v)
            gp.wait()
            pltpu.sync_copy(rows_v, acc_sh.at[dst_v], add=True)
            return 0

        lax.fori_loop(0, nb, body, 0)
        plsc.subcore_barrier()
        pltpu.sync_copy(acc_sh.at[pl.ds(s * rpt, rpt)],
                        out_h.at[pl.ds(c * np_ + s * rpt, rpt)])

    return k(xs, src, dst)


def _decode(P, Q, sl, dl, w, b16):
    n, dh = P.shape
    (e,) = sl.shape
    ept = e // _NW
    B = 400
    assert ept % B == 0 and B % 8 == 0
    nb = ept // B
    nch = dh // 16

    @functools.partial(
        pl.kernel,
        out_type=jax.ShapeDtypeStruct((e,), F32),
        mesh=_sc_mesh(),
        compiler_params=pltpu.CompilerParams(needs_layout_passes=False),
        scratch_types=[
            pltpu.VMEM((B,), jnp.int32),
            pltpu.VMEM((B,), jnp.int32),
            pltpu.VMEM((B, dh), F32),
            pltpu.VMEM((B, dh), F32),
            pltpu.VMEM((dh,), F32),
            pltpu.VMEM((16,), F32),
            pltpu.VMEM((B,), F32),
            pltpu.SemaphoreType.DMA,
            pltpu.SemaphoreType.DMA,
        ],
    )
    def k(p_h, q_h, sl_h, dl_h, w_h, b_h, out_h,
          sl_v, dl_v, p_v, q_v, w_v, b_v, out_v, sem1, sem2):
        c = lax.axis_index("c")
        s = lax.axis_index("s")
        base = (c * _NS + s) * ept
        pltpu.sync_copy(w_h, w_v)
        pltpu.sync_copy(b_h, b_v)
        wregs = [w_v[pl.ds(16 * t, 16)] for t in range(nch)]
        acc0 = b_v[...] * (1.0 / 16.0)

        def batch(bi, _):
            off = base + bi * B
            pltpu.sync_copy(sl_h.at[pl.ds(off, B)], sl_v)
            pltpu.sync_copy(dl_h.at[pl.ds(off, B)], dl_v)
            cp = pltpu.async_copy(p_h.at[sl_v], p_v, sem1)
            cq = pltpu.async_copy(q_h.at[dl_v], q_v, sem2)
            cp.wait()
            cq.wait()

            lanes = lax.iota(jnp.int32, 16)

            def group(gi, _):
                e0 = gi * 16
                ov = jnp.zeros((16,), F32)
                for j in range(16):
                    acc = acc0
                    for t in range(nch):
                        pv = p_v[e0 + j, pl.ds(16 * t, 16)]
                        qv = q_v[e0 + j, pl.ds(16 * t, 16)]
                        acc = acc + jnp.maximum(pv + qv, 0.0) * wregs[t]
                    tot = jnp.sum(acc)
                    ov = jnp.where(lanes == j, jnp.full((16,), tot, F32), ov)
                out_v[pl.ds(e0, 16)] = ov
                return 0

            lax.fori_loop(0, B // 16, group, 0)
            pltpu.sync_copy(out_v, out_h.at[pl.ds(off, B)])
            return 0

        lax.fori_loop(0, nb, batch, 0)

    return k(P, Q, sl, dl, w, b16)


_R = 400


def _tc_encode1(x, W1, d0, d1):
    n, din = x.shape
    dh = W1.shape[1]

    def body(x_ref, w_ref, d0_ref, d1_ref, xs_ref, dinv_ref):
        deg = d0_ref[:, 0:1] + d1_ref[:, 0:1] + 1.0
        dinv = lax.rsqrt(deg)
        xw = jnp.dot(x_ref[...], w_ref[...], preferred_element_type=F32)
        xs_ref[...] = xw * dinv
        dinv_ref[...] = dinv

    return pl.pallas_call(
        body,
        grid=(n // _R,),
        in_specs=[
            pl.BlockSpec((_R, din), lambda i: (i, 0)),
            pl.BlockSpec((din, dh), lambda i: (0, 0)),
            pl.BlockSpec((_R, 128), lambda i: (i, 0)),
            pl.BlockSpec((_R, 128), lambda i: (i, 0)),
        ],
        out_specs=[
            pl.BlockSpec((_R, dh), lambda i: (i, 0)),
            pl.BlockSpec((_R, 1), lambda i: (i, 0)),
        ],
        out_shape=[
            jax.ShapeDtypeStruct((n, dh), F32),
            jax.ShapeDtypeStruct((n, 1), F32),
        ],
    )(x, W1, d0, d1)


def _tc_layer2(s1a, s1b, xs1, dinv, b1r, W2):
    n, dh = xs1.shape
    do = W2.shape[1]

    def body(sa_ref, sb_ref, xs_ref, dinv_ref, b_ref, w_ref, hs_ref):
        dv = dinv_ref[...]
        pre = (sa_ref[...] + sb_ref[...] + xs_ref[...]) * dv + b_ref[...]
        h = jnp.maximum(pre, 0.0)
        hs_ref[...] = jnp.dot(h, w_ref[...], preferred_element_type=F32) * dv

    return pl.pallas_call(
        body,
        grid=(n // _R,),
        in_specs=[
            pl.BlockSpec((_R, dh), lambda i: (i, 0)),
            pl.BlockSpec((_R, dh), lambda i: (i, 0)),
            pl.BlockSpec((_R, dh), lambda i: (i, 0)),
            pl.BlockSpec((_R, 1), lambda i: (i, 0)),
            pl.BlockSpec((1, dh), lambda i: (0, 0)),
            pl.BlockSpec((dh, do), lambda i: (0, 0)),
        ],
        out_specs=pl.BlockSpec((_R, do), lambda i: (i, 0)),
        out_shape=jax.ShapeDtypeStruct((n, do), F32),
    )(s1a, s1b, xs1, dinv, b1r, W2)


def _tc_pq(s2a, s2b, hs2, dinv, b2r, Wm1, bm1r):
    n, dp = hs2.shape
    do, dh = Wm1.shape
    do = do // 2

    def body(sa_ref, sb_ref, hs_ref, dinv_ref, b2_ref, wm_ref, bm_ref,
             p_ref, q_ref):
        z = (sa_ref[...] + sb_ref[...] + hs_ref[...]) * dinv_ref[...] \
            + b2_ref[...]
        zt = z[:, 0:do]
        wm = wm_ref[...]
        p_ref[...] = jnp.dot(zt, wm[0:do], preferred_element_type=F32) \
            + bm_ref[...]
        q_ref[...] = jnp.dot(zt, wm[do:2 * do], preferred_element_type=F32)

    return pl.pallas_call(
        body,
        grid=(n // _R,),
        in_specs=[
            pl.BlockSpec((_R, dp), lambda i: (i, 0)),
            pl.BlockSpec((_R, dp), lambda i: (i, 0)),
            pl.BlockSpec((_R, dp), lambda i: (i, 0)),
            pl.BlockSpec((_R, 1), lambda i: (i, 0)),
            pl.BlockSpec((1, dp), lambda i: (0, 0)),
            pl.BlockSpec((2 * do, dh), lambda i: (0, 0)),
            pl.BlockSpec((1, dh), lambda i: (0, 0)),
        ],
        out_specs=[
            pl.BlockSpec((_R, dh), lambda i: (i, 0)),
            pl.BlockSpec((_R, dh), lambda i: (i, 0)),
        ],
        out_shape=[
            jax.ShapeDtypeStruct((n, dh), F32),
            jax.ShapeDtypeStruct((n, dh), F32),
        ],
    )(s2a, s2b, hs2, dinv, b2r, Wm1, bm1r)


def kernel(x, edge_index, edge_label_index, W1, b1, W2, b2,
           Wm1, bm1, Wm2, bm2):
    n = x.shape[0]
    src = edge_index[0]
    dst = edge_index[1]
    sl = edge_label_index[0]
    dl = edge_label_index[1]

    np_ = _pad_nodes(n)
    dh = W1.shape[1]
    do = W2.shape[1]
    w2p = jnp.pad(W2, ((0, 0), (0, dh - do)))
    b2p = jnp.pad(b2, (0, dh - do)).reshape(1, -1)

    degp = _degree_partials(dst, n)
    xs1, dinv = _tc_encode1(x, W1, degp[:n], degp[np_:np_ + n])
    s1 = _msg_partials(xs1, src, dst)
    hs2 = _tc_layer2(s1[:n], s1[np_:np_ + n], xs1, dinv,
                     b1.reshape(1, -1), w2p)
    s2 = _msg_partials(hs2, src, dst)
    P, Q = _tc_pq(s2[:n], s2[np_:np_ + n], hs2, dinv, b2p,
                  Wm1, bm1.reshape(1, -1))
    out = _decode(P, Q, sl, dl, Wm2.reshape(-1),
                  jnp.broadcast_to(bm2, (16,)))
    return out


# speedup vs baseline: 13.5546x; 1.1726x over previous
import functools

import jax
import jax.numpy as jnp
from jax import lax
from jax.experimental import pallas as pl
from jax.experimental.pallas import tpu as pltpu
from jax.experimental.pallas import tpu_sc as plsc

F32 = jnp.float32

_NC = 2
_NS = 16
_NW = _NC * _NS


def _sc_mesh():
    return plsc.VectorSubcoreMesh(core_axis_name="c", subcore_axis_name="s")


def _pad_nodes(n):
    return ((n + 8 * _NS - 1) // (8 * _NS)) * (8 * _NS)


def _degree_partials(dst, n_nodes):
    (e,) = dst.shape
    ept = e // _NW
    B = 200
    assert ept % B == 0 and B % 8 == 0
    nb = ept // B
    np_ = _pad_nodes(n_nodes)
    rpt = np_ // _NS
    assert (rpt % B) % 8 == 0
    W = 128

    @functools.partial(
        pl.kernel,
        out_type=jax.ShapeDtypeStruct((2 * np_, W), F32),
        mesh=_sc_mesh(),
        scratch_types=[
            pltpu.VMEM((B,), jnp.int32),
            pltpu.VMEM((B, W), F32),
            pltpu.VMEM_SHARED((np_, W), F32),
        ],
    )
    def k(dst_h, out_h, idx_v, ones_v, acc_sh):
        c = lax.axis_index("c")
        s = lax.axis_index("s")

        def fill(val):
            def row(i, _):
                for t in range(W // 16):
                    ones_v[i, pl.ds(16 * t, 16)] = jnp.full((16,), val, F32)
                return 0
            lax.fori_loop(0, B, row, 0)

        fill(0.0)
        for j in range(rpt // B):
            pltpu.sync_copy(ones_v, acc_sh.at[pl.ds(s * rpt + j * B, B)])
        rem = rpt % B
        if rem:
            pltpu.sync_copy(ones_v.at[pl.ds(0, rem)],
                            acc_sh.at[pl.ds(s * rpt + rpt - rem, rem)])
        fill(1.0)
        plsc.subcore_barrier()

        ebase = c * (e // _NC) + s * ept

        def body(b, _):
            off = ebase + b * B
            pltpu.sync_copy(dst_h.at[pl.ds(off, B)], idx_v)
            pltpu.sync_copy(ones_v, acc_sh.at[idx_v], add=True)
            return 0

        lax.fori_loop(0, nb, body, 0)
        plsc.subcore_barrier()
        pltpu.sync_copy(acc_sh.at[pl.ds(s * rpt, rpt)],
                        out_h.at[pl.ds(c * np_ + s * rpt, rpt)])

    return k(dst)


def _msg_partials(xs, src, dst):
    n, d = xs.shape
    (e,) = src.shape
    ept = e // _NW
    B = 80
    assert ept % B == 0 and B % 8 == 0
    nb = ept // B
    assert nb % 2 == 1
    np_ = _pad_nodes(n)
    rpt = np_ // _NS
    assert (rpt % B) % 8 == 0
    nch = d // 16

    @functools.partial(
        pl.kernel,
        out_type=jax.ShapeDtypeStruct((2 * np_, d), F32),
        mesh=_sc_mesh(),
        scratch_types=[
            pltpu.VMEM((B,), jnp.int32),
            pltpu.VMEM((B,), jnp.int32),
            pltpu.VMEM((B,), jnp.int32),
            pltpu.VMEM((B,), jnp.int32),
            pltpu.VMEM((B, d), F32),
            pltpu.VMEM((B, d), F32),
            pltpu.VMEM_SHARED((np_, d), F32),
            pltpu.SemaphoreType.DMA,
            pltpu.SemaphoreType.DMA,
        ],
    )
    def k(xs_h, src_h, dst_h, out_h, src0_v, dst0_v, src1_v, dst1_v,
          rows0_v, rows1_v, acc_sh, sem0, sem1):
        c = lax.axis_index("c")
        s = lax.axis_index("s")

        def zrow(i, _):
            for t in range(nch):
                rows0_v[i, pl.ds(16 * t, 16)] = jnp.zeros((16,), F32)
            return 0

        lax.fori_loop(0, B, zrow, 0)
        r0 = s * rpt
        for j in range(rpt // B):
            pltpu.sync_copy(rows0_v, acc_sh.at[pl.ds(r0 + j * B, B)])
        rem = rpt % B
        if rem:
            pltpu.sync_copy(rows0_v.at[pl.ds(0, rem)],
                            acc_sh.at[pl.ds(r0 + rpt - rem, rem)])
        plsc.subcore_barrier()

        ebase = c * (e // _NC) + s * ept

        def fetch(bi, srcv, rowsv, sem):
            pltpu.sync_copy(src_h.at[pl.ds(ebase + bi * B, B)], srcv)
            pltpu.async_copy(xs_h.at[srcv], rowsv, sem)

        def drain(bi, srcv, dstv, rowsv, sem):
            pltpu.make_async_copy(xs_h.at[srcv], rowsv, sem).wait()
            pltpu.sync_copy(dst_h.at[pl.ds(ebase + bi * B, B)], dstv)
            pltpu.sync_copy(rowsv, acc_sh.at[dstv], add=True)

        fetch(0, src0_v, rows0_v, sem0)

        def body(i, _):
            b0 = 2 * i
            fetch(b0 + 1, src1_v, rows1_v, sem1)
            drain(b0, src0_v, dst0_v, rows0_v, sem0)
            fetch(b0 + 2, src0_v, rows0_v, sem0)
            drain(b0 + 1, src1_v, dst1_v, rows1_v, sem1)
            return 0

        lax.fori_loop(0, nb // 2, body, 0)
        drain(nb - 1, src0_v, dst0_v, rows0_v, sem0)
        plsc.subcore_barrier()
        pltpu.sync_copy(acc_sh.at[pl.ds(s * rpt, rpt)],
                        out_h.at[pl.ds(c * np_ + s * rpt, rpt)])

    return k(xs, src, dst)


def _decode(P, Q, sl, dl, w, b16):
    n, dh = P.shape
    (e,) = sl.shape
    ept = e // _NW
    B = 80
    assert ept % B == 0 and B % 16 == 0
    nb = ept // B
    assert nb % 2 == 1
    nch = dh // 16
    ng = B // 16

    @functools.partial(
        pl.kernel,
        out_type=jax.ShapeDtypeStruct((e,), F32),
        mesh=_sc_mesh(),
        compiler_params=pltpu.CompilerParams(needs_layout_passes=False),
        scratch_types=[
            pltpu.VMEM((B,), jnp.int32),
            pltpu.VMEM((B,), jnp.int32),
            pltpu.VMEM((B,), jnp.int32),
            pltpu.VMEM((B,), jnp.int32),
            pltpu.VMEM((B, dh), F32),
            pltpu.VMEM((B, dh), F32),
            pltpu.VMEM((B, dh), F32),
            pltpu.VMEM((B, dh), F32),
            pltpu.VMEM((dh,), F32),
            pltpu.VMEM((16,), F32),
            pltpu.VMEM((B,), F32),
            pltpu.SemaphoreType.DMA,
            pltpu.SemaphoreType.DMA,
            pltpu.SemaphoreType.DMA,
            pltpu.SemaphoreType.DMA,
        ],
    )
    def k(p_h, q_h, sl_h, dl_h, w_h, b_h, out_h,
          sl0_v, dl0_v, sl1_v, dl1_v, p0_v, q0_v, p1_v, q1_v,
          w_v, b_v, out_v, sp0, sq0, sp1, sq1):
        c = lax.axis_index("c")
        s = lax.axis_index("s")
        base = (c * _NS + s) * ept
        pltpu.sync_copy(w_h, w_v)
        pltpu.sync_copy(b_h, b_v)
        wregs = [w_v[pl.ds(16 * t, 16)] for t in range(nch)]
        acc0 = b_v[...] * (1.0 / 16.0)
        lanes = lax.iota(jnp.int32, 16)

        def fetch(bi, slv, dlv, pv, qv, semp, semq):
            off = base + bi * B
            pltpu.sync_copy(sl_h.at[pl.ds(off, B)], slv)
            pltpu.sync_copy(dl_h.at[pl.ds(off, B)], dlv)
            pltpu.async_copy(p_h.at[slv], pv, semp)
            pltpu.async_copy(q_h.at[dlv], qv, semq)

        def waitbuf(slv, dlv, pv, qv, semp, semq):
            pltpu.make_async_copy(p_h.at[slv], pv, semp).wait()
            pltpu.make_async_copy(q_h.at[dlv], qv, semq).wait()

        def comb(a, b_, d):
            perm = lanes ^ d
            m = (lanes & d) == 0
            a_s = jnp.take_along_axis(a, perm, axis=0)
            b_s = jnp.take_along_axis(b_, perm, axis=0)
            return jnp.where(m, a, b_s) + jnp.where(m, a_s, b_)

        def compute(bi, pv, qv):
            def edge_acc(e0, j):
                acc = acc0
                for t in range(nch):
                    pvv = pv[e0 + j, pl.ds(16 * t, 16)]
                    qvv = qv[e0 + j, pl.ds(16 * t, 16)]
                    acc = acc + jnp.maximum(pvv + qvv, 0.0) * wregs[t]
                return acc

            def group(gi, _):
                e0 = gi * 16
                l1 = [comb(edge_acc(e0, 2 * j), edge_acc(e0, 2 * j + 1), 1)
                      for j in range(8)]
                l2 = [comb(l1[2 * j], l1[2 * j + 1], 2) for j in range(4)]
                l3 = [comb(l2[2 * j], l2[2 * j + 1], 4) for j in range(2)]
                out_v[pl.ds(e0, 16)] = comb(l3[0], l3[1], 8)
                return 0

            lax.fori_loop(0, ng, group, 0)
            pltpu.sync_copy(out_v, out_h.at[pl.ds(base + bi * B, B)])

        fetch(0, sl0_v, dl0_v, p0_v, q0_v, sp0, sq0)

        def body(i, _):
            b0 = 2 * i
            fetch(b0 + 1, sl1_v, dl1_v, p1_v, q1_v, sp1, sq1)
            waitbuf(sl0_v, dl0_v, p0_v, q0_v, sp0, sq0)
            compute(b0, p0_v, q0_v)
            fetch(b0 + 2, sl0_v, dl0_v, p0_v, q0_v, sp0, sq0)
            waitbuf(sl1_v, dl1_v, p1_v, q1_v, sp1, sq1)
            compute(b0 + 1, p1_v, q1_v)
            return 0

        lax.fori_loop(0, nb // 2, body, 0)
        waitbuf(sl0_v, dl0_v, p0_v, q0_v, sp0, sq0)
        compute(nb - 1, p0_v, q0_v)

    return k(P, Q, sl, dl, w, b16)


_R = 400


def _tc_encode1(x, W1, d0, d1):
    n, din = x.shape
    dh = W1.shape[1]

    def body(x_ref, w_ref, d0_ref, d1_ref, xs_ref, dinv_ref):
        deg = d0_ref[:, 0:1] + d1_ref[:, 0:1] + 1.0
        dinv = lax.rsqrt(deg)
        xw = jnp.dot(x_ref[...], w_ref[...], preferred_element_type=F32)
        xs_ref[...] = xw * dinv
        dinv_ref[...] = dinv

    return pl.pallas_call(
        body,
        grid=(n // _R,),
        in_specs=[
            pl.BlockSpec((_R, din), lambda i: (i, 0)),
            pl.BlockSpec((din, dh), lambda i: (0, 0)),
            pl.BlockSpec((_R, 128), lambda i: (i, 0)),
            pl.BlockSpec((_R, 128), lambda i: (i, 0)),
        ],
        out_specs=[
            pl.BlockSpec((_R, dh), lambda i: (i, 0)),
            pl.BlockSpec((_R, 1), lambda i: (i, 0)),
        ],
        out_shape=[
            jax.ShapeDtypeStruct((n, dh), F32),
            jax.ShapeDtypeStruct((n, 1), F32),
        ],
    )(x, W1, d0, d1)


def _tc_layer2(s1a, s1b, xs1, dinv, b1r, W2):
    n, dh = xs1.shape
    do = W2.shape[1]

    def body(sa_ref, sb_ref, xs_ref, dinv_ref, b_ref, w_ref, hs_ref):
        dv = dinv_ref[...]
        pre = (sa_ref[...] + sb_ref[...] + xs_ref[...]) * dv + b_ref[...]
        h = jnp.maximum(pre, 0.0)
        hs_ref[...] = jnp.dot(h, w_ref[...], preferred_element_type=F32) * dv

    return pl.pallas_call(
        body,
        grid=(n // _R,),
        in_specs=[
            pl.BlockSpec((_R, dh), lambda i: (i, 0)),
            pl.BlockSpec((_R, dh), lambda i: (i, 0)),
            pl.BlockSpec((_R, dh), lambda i: (i, 0)),
            pl.BlockSpec((_R, 1), lambda i: (i, 0)),
            pl.BlockSpec((1, dh), lambda i: (0, 0)),
            pl.BlockSpec((dh, do), lambda i: (0, 0)),
        ],
        out_specs=pl.BlockSpec((_R, do), lambda i: (i, 0)),
        out_shape=jax.ShapeDtypeStruct((n, do), F32),
    )(s1a, s1b, xs1, dinv, b1r, W2)


def _tc_pq(s2a, s2b, hs2, dinv, b2r, Wm1, bm1r):
    n, dp = hs2.shape
    do, dh = Wm1.shape
    do = do // 2

    def body(sa_ref, sb_ref, hs_ref, dinv_ref, b2_ref, wm_ref, bm_ref,
             p_ref, q_ref):
        z = (sa_ref[...] + sb_ref[...] + hs_ref[...]) * dinv_ref[...] \
            + b2_ref[...]
        zt = z[:, 0:do]
        wm = wm_ref[...]
        p_ref[...] = jnp.dot(zt, wm[0:do], preferred_element_type=F32) \
            + bm_ref[...]
        q_ref[...] = jnp.dot(zt, wm[do:2 * do], preferred_element_type=F32)

    return pl.pallas_call(
        body,
        grid=(n // _R,),
        in_specs=[
            pl.BlockSpec((_R, dp), lambda i: (i, 0)),
            pl.BlockSpec((_R, dp), lambda i: (i, 0)),
            pl.BlockSpec((_R, dp), lambda i: (i, 0)),
            pl.BlockSpec((_R, 1), lambda i: (i, 0)),
            pl.BlockSpec((1, dp), lambda i: (0, 0)),
            pl.BlockSpec((2 * do, dh), lambda i: (0, 0)),
            pl.BlockSpec((1, dh), lambda i: (0, 0)),
        ],
        out_specs=[
            pl.BlockSpec((_R, dh), lambda i: (i, 0)),
            pl.BlockSpec((_R, dh), lambda i: (i, 0)),
        ],
        out_shape=[
            jax.ShapeDtypeStruct((n, dh), F32),
            jax.ShapeDtypeStruct((n, dh), F32),
        ],
    )(s2a, s2b, hs2, dinv, b2r, Wm1, bm1r)


def kernel(x, edge_index, edge_label_index, W1, b1, W2, b2,
           Wm1, bm1, Wm2, bm2):
    n = x.shape[0]
    src = edge_index[0]
    dst = edge_index[1]
    sl = edge_label_index[0]
    dl = edge_label_index[1]

    np_ = _pad_nodes(n)
    dh = W1.shape[1]
    do = W2.shape[1]
    w2p = jnp.pad(W2, ((0, 0), (0, dh - do)))
    b2p = jnp.pad(b2, (0, dh - do)).reshape(1, -1)

    degp = _degree_partials(dst, n)
    xs1, dinv = _tc_encode1(x, W1, degp[:n], degp[np_:np_ + n])
    s1 = _msg_partials(xs1, src, dst)
    hs2 = _tc_layer2(s1[:n], s1[np_:np_ + n], xs1, dinv,
                     b1.reshape(1, -1), w2p)
    s2 = _msg_partials(hs2, src, dst)
    P, Q = _tc_pq(s2[:n], s2[np_:np_ + n], hs2, dinv, b2p,
                  Wm1, bm1.reshape(1, -1))
    out = _decode(P, Q, sl, dl, Wm2.reshape(-1),
                  jnp.broadcast_to(bm2, (16,)))
    return out


# speedup vs baseline: 17.2858x; 1.2753x over previous
import functools

import jax
import jax.numpy as jnp
from jax import lax
from jax.experimental import pallas as pl
from jax.experimental.pallas import tpu as pltpu
from jax.experimental.pallas import tpu_sc as plsc

F32 = jnp.float32

_NC = 2
_NS = 16
_NW = _NC * _NS


def _sc_mesh():
    return plsc.VectorSubcoreMesh(core_axis_name="c", subcore_axis_name="s")


def _pad_nodes(n):
    return ((n + 8 * _NS - 1) // (8 * _NS)) * (8 * _NS)


def _degree_partials(dst, n_nodes):
    (e,) = dst.shape
    ept = e // _NW
    B = 200
    assert ept % B == 0 and B % 8 == 0
    nb = ept // B
    np_ = _pad_nodes(n_nodes)
    rpt = np_ // _NS
    assert (rpt % B) % 8 == 0
    W = 128

    @functools.partial(
        pl.kernel,
        out_type=jax.ShapeDtypeStruct((2 * np_, W), F32),
        mesh=_sc_mesh(),
        scratch_types=[
            pltpu.VMEM((B,), jnp.int32),
            pltpu.VMEM((B,), jnp.int32),
            pltpu.VMEM((B, W), F32),
            pltpu.VMEM_SHARED((np_, W), F32),
            pltpu.SemaphoreType.DMA,
            pltpu.SemaphoreType.DMA,
        ],
    )
    def k(dst_h, out_h, idx0_v, idx1_v, ones_v, acc_sh, s0, s1):
        c = lax.axis_index("c")
        s = lax.axis_index("s")

        def fill(val):
            def row(i, _):
                for t in range(W // 16):
                    ones_v[i, pl.ds(16 * t, 16)] = jnp.full((16,), val, F32)
                return 0
            lax.fori_loop(0, B, row, 0)

        fill(0.0)
        for j in range(rpt // B):
            pltpu.sync_copy(ones_v, acc_sh.at[pl.ds(s * rpt + j * B, B)])
        rem = rpt % B
        if rem:
            pltpu.sync_copy(ones_v.at[pl.ds(0, rem)],
                            acc_sh.at[pl.ds(s * rpt + rpt - rem, rem)])
        fill(1.0)
        plsc.subcore_barrier()

        ebase = c * (e // _NC) + s * ept

        def fetch(bi, v, sem):
            off = ebase + jnp.minimum(bi, nb - 1) * B
            pltpu.async_copy(dst_h.at[pl.ds(off, B)], v, sem)

        def wait(v, sem):
            pltpu.make_async_copy(dst_h.at[pl.ds(ebase, B)], v, sem).wait()

        fetch(0, idx0_v, s0)

        def body(i, _):
            b0 = 2 * i
            fetch(b0 + 1, idx1_v, s1)
            wait(idx0_v, s0)
            pltpu.sync_copy(ones_v, acc_sh.at[idx0_v], add=True)
            fetch(b0 + 2, idx0_v, s0)
            wait(idx1_v, s1)
            pltpu.sync_copy(ones_v, acc_sh.at[idx1_v], add=True)
            return 0

        assert nb % 2 == 0
        lax.fori_loop(0, nb // 2, body, 0)
        wait(idx0_v, s0)
        plsc.subcore_barrier()
        pltpu.sync_copy(acc_sh.at[pl.ds(s * rpt, rpt)],
                        out_h.at[pl.ds(c * np_ + s * rpt, rpt)])

    return k(dst)


def _msg_partials(xs, src, dst):
    n, d = xs.shape
    (e,) = src.shape
    ept = e // _NW
    B = 80
    assert ept % B == 0 and B % 8 == 0
    nb = ept // B
    assert nb % 2 == 1
    np_ = _pad_nodes(n)
    rpt = np_ // _NS
    assert (rpt % B) % 8 == 0
    nch = d // 16

    @functools.partial(
        pl.kernel,
        out_type=jax.ShapeDtypeStruct((2 * np_, d), F32),
        mesh=_sc_mesh(),
        scratch_types=[
            pltpu.VMEM((B,), jnp.int32),
            pltpu.VMEM((B,), jnp.int32),
            pltpu.VMEM((B,), jnp.int32),
            pltpu.VMEM((B,), jnp.int32),
            pltpu.VMEM((B, d), F32),
            pltpu.VMEM((B, d), F32),
            pltpu.VMEM_SHARED((np_, d), F32),
            pltpu.SemaphoreType.DMA,
            pltpu.SemaphoreType.DMA,
            pltpu.SemaphoreType.DMA,
            pltpu.SemaphoreType.DMA,
            pltpu.SemaphoreType.DMA,
            pltpu.SemaphoreType.DMA,
        ],
    )
    def k(xs_h, src_h, dst_h, out_h, src0_v, dst0_v, src1_v, dst1_v,
          rows0_v, rows1_v, acc_sh, sem0, sem1, si0, sd0, si1, sd1):
        c = lax.axis_index("c")
        s = lax.axis_index("s")

        def zrow(i, _):
            for t in range(nch):
                rows0_v[i, pl.ds(16 * t, 16)] = jnp.zeros((16,), F32)
            return 0

        lax.fori_loop(0, B, zrow, 0)
        r0 = s * rpt
        for j in range(rpt // B):
            pltpu.sync_copy(rows0_v, acc_sh.at[pl.ds(r0 + j * B, B)])
        rem = rpt % B
        if rem:
            pltpu.sync_copy(rows0_v.at[pl.ds(0, rem)],
                            acc_sh.at[pl.ds(r0 + rpt - rem, rem)])
        plsc.subcore_barrier()

        ebase = c * (e // _NC) + s * ept

        def fetch_idx(bi, srcv, dstv, sis, sid):
            off = ebase + jnp.minimum(bi, nb - 1) * B
            pltpu.async_copy(src_h.at[pl.ds(off, B)], srcv, sis)
            pltpu.async_copy(dst_h.at[pl.ds(off, B)], dstv, sid)

        def wait_idx(srcv, dstv, sis, sid):
            pltpu.make_async_copy(src_h.at[pl.ds(ebase, B)], srcv, sis).wait()
            pltpu.make_async_copy(dst_h.at[pl.ds(ebase, B)], dstv, sid).wait()

        def wait_rows(srcv, rowsv, sem):
            pltpu.make_async_copy(xs_h.at[srcv], rowsv, sem).wait()

        fetch_idx(0, src0_v, dst0_v, si0, sd0)
        wait_idx(src0_v, dst0_v, si0, sd0)
        pltpu.async_copy(xs_h.at[src0_v], rows0_v, sem0)
        fetch_idx(1, src1_v, dst1_v, si1, sd1)

        def body(i, _):
            b0 = 2 * i
            wait_idx(src1_v, dst1_v, si1, sd1)
            pltpu.async_copy(xs_h.at[src1_v], rows1_v, sem1)
            wait_rows(src0_v, rows0_v, sem0)
            pltpu.sync_copy(rows0_v, acc_sh.at[dst0_v], add=True)
            fetch_idx(b0 + 2, src0_v, dst0_v, si0, sd0)
            wait_rows(src1_v, rows1_v, sem1)
            pltpu.sync_copy(rows1_v, acc_sh.at[dst1_v], add=True)
            wait_idx(src0_v, dst0_v, si0, sd0)
            pltpu.async_copy(xs_h.at[src0_v], rows0_v, sem0)
            fetch_idx(b0 + 3, src1_v, dst1_v, si1, sd1)
            return 0

        lax.fori_loop(0, nb // 2, body, 0)
        wait_rows(src0_v, rows0_v, sem0)
        pltpu.sync_copy(rows0_v, acc_sh.at[dst0_v], add=True)
        wait_idx(src1_v, dst1_v, si1, sd1)
        plsc.subcore_barrier()
        pltpu.sync_copy(acc_sh.at[pl.ds(s * rpt, rpt)],
                        out_h.at[pl.ds(c * np_ + s * rpt, rpt)])

    return k(xs, src, dst)


def _decode(P, Q, sl, dl, w, b16):
    n, dh = P.shape
    (e,) = sl.shape
    ept = e // _NW
    B = 80
    assert ept % B == 0 and B % 16 == 0
    nb = ept // B
    assert nb % 2 == 1
    nch = dh // 16
    ng = B // 16

    @functools.partial(
        pl.kernel,
        out_type=jax.ShapeDtypeStruct((e,), F32),
        mesh=_sc_mesh(),
        compiler_params=pltpu.CompilerParams(needs_layout_passes=False),
        scratch_types=[
            pltpu.VMEM((B,), jnp.int32),
            pltpu.VMEM((B,), jnp.int32),
            pltpu.VMEM((B,), jnp.int32),
            pltpu.VMEM((B,), jnp.int32),
            pltpu.VMEM((B, dh), F32),
            pltpu.VMEM((B, dh), F32),
            pltpu.VMEM((B, dh), F32),
            pltpu.VMEM((B, dh), F32),
            pltpu.VMEM((dh,), F32),
            pltpu.VMEM((16,), F32),
            pltpu.VMEM((B,), F32),
            pltpu.SemaphoreType.DMA,
            pltpu.SemaphoreType.DMA,
            pltpu.SemaphoreType.DMA,
            pltpu.SemaphoreType.DMA,
            pltpu.SemaphoreType.DMA,
            pltpu.SemaphoreType.DMA,
            pltpu.SemaphoreType.DMA,
            pltpu.SemaphoreType.DMA,
        ],
    )
    def k(p_h, q_h, sl_h, dl_h, w_h, b_h, out_h,
          sl0_v, dl0_v, sl1_v, dl1_v, p0_v, q0_v, p1_v, q1_v,
          w_v, b_v, out_v, sp0, sq0, sp1, sq1, ss0, se0, ss1, se1):
        c = lax.axis_index("c")
        s = lax.axis_index("s")
        base = (c * _NS + s) * ept
        pltpu.sync_copy(w_h, w_v)
        pltpu.sync_copy(b_h, b_v)
        wregs = [w_v[pl.ds(16 * t, 16)] for t in range(nch)]
        acc0 = b_v[...] * (1.0 / 16.0)
        lanes = lax.iota(jnp.int32, 16)

        def fetch_idx(bi, slv, dlv, sis, sid):
            off = base + jnp.minimum(bi, nb - 1) * B
            pltpu.async_copy(sl_h.at[pl.ds(off, B)], slv, sis)
            pltpu.async_copy(dl_h.at[pl.ds(off, B)], dlv, sid)

        def wait_idx(slv, dlv, sis, sid):
            pltpu.make_async_copy(sl_h.at[pl.ds(base, B)], slv, sis).wait()
            pltpu.make_async_copy(dl_h.at[pl.ds(base, B)], dlv, sid).wait()

        def gathers(slv, dlv, pv, qv, semp, semq):
            pltpu.async_copy(p_h.at[slv], pv, semp)
            pltpu.async_copy(q_h.at[dlv], qv, semq)

        def waitbuf(slv, dlv, pv, qv, semp, semq):
            pltpu.make_async_copy(p_h.at[slv], pv, semp).wait()
            pltpu.make_async_copy(q_h.at[dlv], qv, semq).wait()

        def comb(a, b_, d):
            perm = lanes ^ d
            m = (lanes & d) == 0
            a_s = jnp.take_along_axis(a, perm, axis=0)
            b_s = jnp.take_along_axis(b_, perm, axis=0)
            return jnp.where(m, a, b_s) + jnp.where(m, a_s, b_)

        def compute(bi, pv, qv):
            def edge_acc(e0, j):
                acc = acc0
                for t in range(nch):
                    pvv = pv[e0 + j, pl.ds(16 * t, 16)]
                    qvv = qv[e0 + j, pl.ds(16 * t, 16)]
                    acc = acc + jnp.maximum(pvv + qvv, 0.0) * wregs[t]
                return acc

            def group(gi, _):
                e0 = gi * 16
                l1 = [comb(edge_acc(e0, 2 * j), edge_acc(e0, 2 * j + 1), 1)
                      for j in range(8)]
                l2 = [comb(l1[2 * j], l1[2 * j + 1], 2) for j in range(4)]
                l3 = [comb(l2[2 * j], l2[2 * j + 1], 4) for j in range(2)]
                out_v[pl.ds(e0, 16)] = comb(l3[0], l3[1], 8)
                return 0

            lax.fori_loop(0, ng, group, 0)
            pltpu.sync_copy(out_v, out_h.at[pl.ds(base + bi * B, B)])

        fetch_idx(0, sl0_v, dl0_v, ss0, se0)
        wait_idx(sl0_v, dl0_v, ss0, se0)
        gathers(sl0_v, dl0_v, p0_v, q0_v, sp0, sq0)
        fetch_idx(1, sl1_v, dl1_v, ss1, se1)

        def body(i, _):
            b0 = 2 * i
            wait_idx(sl1_v, dl1_v, ss1, se1)
            gathers(sl1_v, dl1_v, p1_v, q1_v, sp1, sq1)
            waitbuf(sl0_v, dl0_v, p0_v, q0_v, sp0, sq0)
            fetch_idx(b0 + 2, sl0_v, dl0_v, ss0, se0)
            compute(b0, p0_v, q0_v)
            wait_idx(sl0_v, dl0_v, ss0, se0)
            gathers(sl0_v, dl0_v, p0_v, q0_v, sp0, sq0)
            waitbuf(sl1_v, dl1_v, p1_v, q1_v, sp1, sq1)
            fetch_idx(b0 + 3, sl1_v, dl1_v, ss1, se1)
            compute(b0 + 1, p1_v, q1_v)
            return 0

        lax.fori_loop(0, nb // 2, body, 0)
        waitbuf(sl0_v, dl0_v, p0_v, q0_v, sp0, sq0)
        compute(nb - 1, p0_v, q0_v)
        wait_idx(sl1_v, dl1_v, ss1, se1)

    return k(P, Q, sl, dl, w, b16)


_R = 400


def _tc_encode1(x, W1, d0, d1):
    n, din = x.shape
    dh = W1.shape[1]

    def body(x_ref, w_ref, d0_ref, d1_ref, xs_ref, dinv_ref):
        deg = d0_ref[:, 0:1] + d1_ref[:, 0:1] + 1.0
        dinv = lax.rsqrt(deg)
        xw = jnp.dot(x_ref[...], w_ref[...], preferred_element_type=F32)
        xs_ref[...] = xw * dinv
        dinv_ref[...] = dinv

    return pl.pallas_call(
        body,
        grid=(n // _R,),
        in_specs=[
            pl.BlockSpec((_R, din), lambda i: (i, 0)),
            pl.BlockSpec((din, dh), lambda i: (0, 0)),
            pl.BlockSpec((_R, 128), lambda i: (i, 0)),
            pl.BlockSpec((_R, 128), lambda i: (i, 0)),
        ],
        out_specs=[
            pl.BlockSpec((_R, dh), lambda i: (i, 0)),
            pl.BlockSpec((_R, 1), lambda i: (i, 0)),
        ],
        out_shape=[
            jax.ShapeDtypeStruct((n, dh), F32),
            jax.ShapeDtypeStruct((n, 1), F32),
        ],
    )(x, W1, d0, d1)


def _tc_layer2(s1a, s1b, xs1, dinv, b1r, W2):
    n, dh = xs1.shape
    do = W2.shape[1]

    def body(sa_ref, sb_ref, xs_ref, dinv_ref, b_ref, w_ref, hs_ref):
        dv = dinv_ref[...]
        pre = (sa_ref[...] + sb_ref[...] + xs_ref[...]) * dv + b_ref[...]
        h = jnp.maximum(pre, 0.0)
        hs_ref[...] = jnp.dot(h, w_ref[...], preferred_element_type=F32) * dv

    return pl.pallas_call(
        body,
        grid=(n // _R,),
        in_specs=[
            pl.BlockSpec((_R, dh), lambda i: (i, 0)),
            pl.BlockSpec((_R, dh), lambda i: (i, 0)),
            pl.BlockSpec((_R, dh), lambda i: (i, 0)),
            pl.BlockSpec((_R, 1), lambda i: (i, 0)),
            pl.BlockSpec((1, dh), lambda i: (0, 0)),
            pl.BlockSpec((dh, do), lambda i: (0, 0)),
        ],
        out_specs=pl.BlockSpec((_R, do), lambda i: (i, 0)),
        out_shape=jax.ShapeDtypeStruct((n, do), F32),
    )(s1a, s1b, xs1, dinv, b1r, W2)


def _tc_pq(s2a, s2b, hs2, dinv, b2r, Wm1, bm1r):
    n, dp = hs2.shape
    do, dh = Wm1.shape
    do = do // 2

    def body(sa_ref, sb_ref, hs_ref, dinv_ref, b2_ref, wm_ref, bm_ref,
             p_ref, q_ref):
        z = (sa_ref[...] + sb_ref[...] + hs_ref[...]) * dinv_ref[...] \
            + b2_ref[...]
        zt = z[:, 0:do]
        wm = wm_ref[...]
        p_ref[...] = jnp.dot(zt, wm[0:do], preferred_element_type=F32) \
            + bm_ref[...]
        q_ref[...] = jnp.dot(zt, wm[do:2 * do], preferred_element_type=F32)

    return pl.pallas_call(
        body,
        grid=(n // _R,),
        in_specs=[
            pl.BlockSpec((_R, dp), lambda i: (i, 0)),
            pl.BlockSpec((_R, dp), lambda i: (i, 0)),
            pl.BlockSpec((_R, dp), lambda i: (i, 0)),
            pl.BlockSpec((_R, 1), lambda i: (i, 0)),
            pl.BlockSpec((1, dp), lambda i: (0, 0)),
            pl.BlockSpec((2 * do, dh), lambda i: (0, 0)),
            pl.BlockSpec((1, dh), lambda i: (0, 0)),
        ],
        out_specs=[
            pl.BlockSpec((_R, dh), lambda i: (i, 0)),
            pl.BlockSpec((_R, dh), lambda i: (i, 0)),
        ],
        out_shape=[
            jax.ShapeDtypeStruct((n, dh), F32),
            jax.ShapeDtypeStruct((n, dh), F32),
        ],
    )(s2a, s2b, hs2, dinv, b2r, Wm1, bm1r)


def kernel(x, edge_index, edge_label_index, W1, b1, W2, b2,
           Wm1, bm1, Wm2, bm2):
    n = x.shape[0]
    src = edge_index[0]
    dst = edge_index[1]
    sl = edge_label_index[0]
    dl = edge_label_index[1]

    np_ = _pad_nodes(n)
    dh = W1.shape[1]
    do = W2.shape[1]
    w2p = jnp.pad(W2, ((0, 0), (0, dh - do)))
    b2p = jnp.pad(b2, (0, dh - do)).reshape(1, -1)

    degp = _degree_partials(dst, n)
    xs1, dinv = _tc_encode1(x, W1, degp[:n], degp[np_:np_ + n])
    s1 = _msg_partials(xs1, src, dst)
    hs2 = _tc_layer2(s1[:n], s1[np_:np_ + n], xs1, dinv,
                     b1.reshape(1, -1), w2p)
    s2 = _msg_partials(hs2, src, dst)
    P, Q = _tc_pq(s2[:n], s2[np_:np_ + n], hs2, dinv, b2p,
                  Wm1, bm1.reshape(1, -1))
    out = _decode(P, Q, sl, dl, Wm2.reshape(-1),
                  jnp.broadcast_to(bm2, (16,)))
    return out
